# baseline (speedup 1.0000x reference)
import functools

import jax
import jax.numpy as jnp
from jax import lax
from jax.experimental import pallas as pl
from jax.experimental.pallas import tpu as pltpu
from jax.experimental.pallas import tpu_sc as plsc

_VOCAB = 100000
_EMBED = 64
_CTX = 50
_HID = 128
_CTX_PAD = 64

_BLK = 5120
_NBLK = -(-_VOCAB // _BLK)
_PAD_N = _NBLK * _BLK
_NEG = -1e30


@functools.cache
def _make_sc_gather():
    @functools.partial(
        pl.kernel,
        out_type=jax.ShapeDtypeStruct((_CTX_PAD, _EMBED), jnp.float32),
        mesh=plsc.VectorSubcoreMesh(core_axis_name="c", subcore_axis_name="s"),
        scratch_types=[
            pltpu.VMEM((_CTX_PAD,), jnp.int32),
            pltpu.VMEM((_CTX_PAD, _EMBED), jnp.float32),
            pltpu.SemaphoreType.DMA,
        ],
        compiler_params=pltpu.CompilerParams(use_tc_tiling_on_sc=False),
    )
    def _sc_gather(idx_hbm, table_hbm, out_hbm, idx_v, rows_v, sem):
        cid = lax.axis_index("c")
        sid = lax.axis_index("s")

        @pl.when((cid == 0) & (sid == 0))
        def _():
            pltpu.sync_copy(idx_hbm, idx_v)
            pltpu.async_copy(table_hbm.at[idx_v], rows_v, sem).wait()
            pltpu.sync_copy(rows_v, out_hbm)

    return _sc_gather


def _mlp_body(emb_ref, w1_ref, b1_ref, w2_ref, b2_ref, out_ref,
              h_ref, vm_ref, vs_ref):
    i = pl.program_id(0)

    @pl.when(i == 0)
    def _init():
        h = lax.dot_general(emb_ref[...], w1_ref[...],
                            (((1,), (1,)), ((), ())),
                            preferred_element_type=jnp.float32)
        h_ref[...] = jnp.maximum(h + b1_ref[...], 0.0)
        vm_ref[...] = jnp.full_like(vm_ref, _NEG)
        vs_ref[...] = jnp.zeros_like(vs_ref)

    logits = lax.dot_general(h_ref[...], w2_ref[...],
                             (((1,), (1,)), ((), ())),
                             preferred_element_type=jnp.float32)
    logits = logits + b2_ref[...]
    out_ref[:, pl.ds(i * _BLK, _BLK)] = logits

    col = i * _BLK + lax.broadcasted_iota(jnp.int32, (1, _BLK), 1)
    lg = jnp.where(col < _VOCAB, logits, _NEG)
    bmax = jnp.max(lg, axis=1, keepdims=True)
    psum = jnp.sum(jnp.exp(lg - bmax), axis=1, keepdims=True)
    bm = jnp.broadcast_to(bmax, (1, 128))
    ps = jnp.broadcast_to(psum, (1, 128))
    vm_old = vm_ref[...]
    vm_new = jnp.maximum(vm_old, bm)
    vs_ref[...] = (vs_ref[...] * jnp.exp(vm_old - vm_new)
                   + ps * jnp.exp(bm - vm_new))
    vm_ref[...] = vm_new

    @pl.when(i == _NBLK - 1)
    def _fin():
        lse = (vm_ref[0:1, 0:1]
               + jnp.log(vs_ref[0:1, 0:1]))
        out_ref[...] = out_ref[...] - lse


def _mlp_call(embeds, W1, b1_2d, W2, b2_2d, interpret=False):
    return pl.pallas_call(
        _mlp_body,
        grid=(_NBLK,),
        in_specs=[
            pl.BlockSpec((1, _CTX * _EMBED), lambda i: (0, 0)),
            pl.BlockSpec((_HID, _CTX * _EMBED), lambda i: (0, 0)),
            pl.BlockSpec((1, _HID), lambda i: (0, 0)),
            pl.BlockSpec((_BLK, _HID), lambda i: (i, 0)),
            pl.BlockSpec((1, _BLK), lambda i: (0, i)),
        ],
        out_specs=pl.BlockSpec((1, _PAD_N), lambda i: (0, 0)),
        out_shape=jax.ShapeDtypeStruct((1, _VOCAB), jnp.float32),
        scratch_shapes=[
            pltpu.VMEM((1, _HID), jnp.float32),
            pltpu.VMEM((1, 128), jnp.float32),
            pltpu.VMEM((1, 128), jnp.float32),
        ],
        interpret=interpret,
    )(embeds, W1, b1_2d, W2, b2_2d)


def kernel(inputs, emb_table, W1, b1, W2, b2):
    idx = jnp.pad(inputs.astype(jnp.int32), (0, _CTX_PAD - _CTX))
    rows = _make_sc_gather()(idx, emb_table)
    embeds = rows[:_CTX].reshape(1, _CTX * _EMBED)
    return _mlp_call(embeds, W1, b1.reshape(1, _HID), W2,
                     b2.reshape(1, _VOCAB))


# speedup vs baseline: 1.0682x; 1.0682x over previous
import functools

import jax
import jax.numpy as jnp
from jax import lax
from jax.experimental import pallas as pl
from jax.experimental.pallas import tpu as pltpu
from jax.experimental.pallas import tpu_sc as plsc

_VOCAB = 100000
_EMBED = 64
_CTX = 50
_HID = 128
_CTX_PAD = 64

_BLK = 12800
_NBLK = -(-_VOCAB // _BLK)
_PAD_N = _NBLK * _BLK
_NEG = -1e30


@functools.cache
def _make_sc_gather():
    @functools.partial(
        pl.kernel,
        out_type=jax.ShapeDtypeStruct((_CTX_PAD, _EMBED), jnp.float32),
        mesh=plsc.VectorSubcoreMesh(core_axis_name="c", subcore_axis_name="s"),
        scratch_types=[
            pltpu.VMEM((_CTX_PAD,), jnp.int32),
            pltpu.VMEM((_CTX_PAD, _EMBED), jnp.float32),
            pltpu.SemaphoreType.DMA,
        ],
        compiler_params=pltpu.CompilerParams(use_tc_tiling_on_sc=False),
    )
    def _sc_gather(idx_hbm, table_hbm, out_hbm, idx_v, rows_v, sem):
        cid = lax.axis_index("c")
        sid = lax.axis_index("s")

        @pl.when((cid == 0) & (sid == 0))
        def _():
            pltpu.sync_copy(idx_hbm, idx_v)
            pltpu.async_copy(table_hbm.at[idx_v], rows_v, sem).wait()
            pltpu.sync_copy(rows_v, out_hbm)

    return _sc_gather


def _mlp_body(emb_ref, w1_ref, b1_ref, w2_ref, b2_ref, out_ref,
              h_ref, vm_ref, vs_ref):
    i = pl.program_id(0)

    @pl.when(i == 0)
    def _init():
        h = lax.dot_general(emb_ref[...], w1_ref[...],
                            (((1,), (1,)), ((), ())),
                            preferred_element_type=jnp.float32)
        h_ref[...] = jnp.maximum(h + b1_ref[...], 0.0)
        vm_ref[...] = jnp.full_like(vm_ref, _NEG)
        vs_ref[...] = jnp.zeros_like(vs_ref)

    logits = lax.dot_general(h_ref[...], w2_ref[...],
                             (((1,), (1,)), ((), ())),
                             preferred_element_type=jnp.float32)
    logits = logits + b2_ref[...]
    out_ref[:, pl.ds(i * _BLK, _BLK)] = logits

    col = i * _BLK + lax.broadcasted_iota(jnp.int32, (1, _BLK), 1)
    lg = jnp.where(col < _VOCAB, logits, _NEG)
    bmax = jnp.max(lg, axis=1, keepdims=True)
    psum = jnp.sum(jnp.exp(lg - bmax), axis=1, keepdims=True)
    bm = jnp.broadcast_to(bmax, (1, 128))
    ps = jnp.broadcast_to(psum, (1, 128))
    vm_old = vm_ref[...]
    vm_new = jnp.maximum(vm_old, bm)
    vs_ref[...] = (vs_ref[...] * jnp.exp(vm_old - vm_new)
                   + ps * jnp.exp(bm - vm_new))
    vm_ref[...] = vm_new

    @pl.when(i == _NBLK - 1)
    def _fin():
        lse = (vm_ref[0:1, 0:1]
               + jnp.log(vs_ref[0:1, 0:1]))
        out_ref[...] = out_ref[...] - lse


def _mlp_call(embeds, W1, b1_2d, W2, b2_2d, interpret=False):
    return pl.pallas_call(
        _mlp_body,
        grid=(_NBLK,),
        in_specs=[
            pl.BlockSpec((1, _CTX * _EMBED), lambda i: (0, 0)),
            pl.BlockSpec((_HID, _CTX * _EMBED), lambda i: (0, 0)),
            pl.BlockSpec((1, _HID), lambda i: (0, 0)),
            pl.BlockSpec((_BLK, _HID), lambda i: (i, 0)),
            pl.BlockSpec((1, _BLK), lambda i: (0, i)),
        ],
        out_specs=pl.BlockSpec((1, _PAD_N), lambda i: (0, 0)),
        out_shape=jax.ShapeDtypeStruct((1, _VOCAB), jnp.float32),
        scratch_shapes=[
            pltpu.VMEM((1, _HID), jnp.float32),
            pltpu.VMEM((1, 128), jnp.float32),
            pltpu.VMEM((1, 128), jnp.float32),
        ],
        interpret=interpret,
    )(embeds, W1, b1_2d, W2, b2_2d)


def kernel(inputs, emb_table, W1, b1, W2, b2):
    idx = jnp.pad(inputs.astype(jnp.int32), (0, _CTX_PAD - _CTX))
    rows = _make_sc_gather()(idx, emb_table)
    embeds = rows[:_CTX].reshape(1, _CTX * _EMBED)
    return _mlp_call(embeds, W1, b1.reshape(1, _HID), W2,
                     b2.reshape(1, _VOCAB))


# speedup vs baseline: 1.0682x; 1.0000x over previous
import functools

import jax
import jax.numpy as jnp
from jax import lax
from jax.experimental import pallas as pl
from jax.experimental.pallas import tpu as pltpu
from jax.experimental.pallas import tpu_sc as plsc

_VOCAB = 100000
_EMBED = 64
_CTX = 50
_HID = 128
_CTX_PAD = 64

_NSTREAM = 4
_BLK = 3200
_NBLK = 8
_PAD_N = _NSTREAM * _NBLK * _BLK
_NEG = -1e30


@functools.cache
def _make_sc_gather():
    @functools.partial(
        pl.kernel,
        out_type=jax.ShapeDtypeStruct((_CTX_PAD, _EMBED), jnp.float32),
        mesh=plsc.VectorSubcoreMesh(core_axis_name="c", subcore_axis_name="s"),
        scratch_types=[
            pltpu.VMEM((_CTX_PAD,), jnp.int32),
            pltpu.VMEM((_CTX_PAD, _EMBED), jnp.float32),
            pltpu.SemaphoreType.DMA,
        ],
        compiler_params=pltpu.CompilerParams(use_tc_tiling_on_sc=False),
    )
    def _sc_gather(idx_hbm, table_hbm, out_hbm, idx_v, rows_v, sem):
        cid = lax.axis_index("c")
        sid = lax.axis_index("s")

        @pl.when((cid == 0) & (sid == 0))
        def _():
            pltpu.sync_copy(idx_hbm, idx_v)
            pltpu.async_copy(table_hbm.at[idx_v], rows_v, sem).wait()
            pltpu.sync_copy(rows_v, out_hbm)

    return _sc_gather


def _mlp_body(emb_ref, w1_ref, b1_ref, *refs):
    w2_refs = refs[:_NSTREAM]
    b2_refs = refs[_NSTREAM:2 * _NSTREAM]
    out_ref = refs[2 * _NSTREAM]
    h_ref, vm_ref, vs_ref = refs[2 * _NSTREAM + 1:]
    i = pl.program_id(0)

    @pl.when(i == 0)
    def _init():
        h = lax.dot_general(emb_ref[...], w1_ref[...],
                            (((1,), (1,)), ((), ())),
                            preferred_element_type=jnp.float32)
        h_ref[...] = jnp.maximum(h + b1_ref[...], 0.0)
        vm_ref[...] = jnp.full_like(vm_ref, _NEG)
        vs_ref[...] = jnp.zeros_like(vs_ref)

    for s in range(_NSTREAM):
        base = (s * _NBLK + i) * _BLK
        logits = lax.dot_general(h_ref[...], w2_refs[s][...],
                                 (((1,), (1,)), ((), ())),
                                 preferred_element_type=jnp.float32)
        logits = logits + b2_refs[s][...]
        out_ref[:, pl.ds(base, _BLK)] = logits

        col = base + lax.broadcasted_iota(jnp.int32, (1, _BLK), 1)
        lg = jnp.where(col < _VOCAB, logits, _NEG)
        bmax = jnp.max(lg, axis=1, keepdims=True)
        psum = jnp.sum(jnp.exp(lg - bmax), axis=1, keepdims=True)
        bm = jnp.broadcast_to(bmax, (1, 128))
        ps = jnp.broadcast_to(psum, (1, 128))
        vm_old = vm_ref[...]
        vm_new = jnp.maximum(vm_old, bm)
        vs_ref[...] = (vs_ref[...] * jnp.exp(vm_old - vm_new)
                       + ps * jnp.exp(bm - vm_new))
        vm_ref[...] = vm_new

    @pl.when(i == _NBLK - 1)
    def _fin():
        lse = (vm_ref[0:1, 0:1]
               + jnp.log(vs_ref[0:1, 0:1]))
        out_ref[...] = out_ref[...] - lse


def _mlp_call(embeds, W1, b1_2d, W2, b2_2d, interpret=False):
    w2_specs = [
        pl.BlockSpec((_BLK, _HID), lambda i, s=s: (s * _NBLK + i, 0))
        for s in range(_NSTREAM)
    ]
    b2_specs = [
        pl.BlockSpec((1, _BLK), lambda i, s=s: (0, s * _NBLK + i))
        for s in range(_NSTREAM)
    ]
    return pl.pallas_call(
        _mlp_body,
        grid=(_NBLK,),
        in_specs=[
            pl.BlockSpec((1, _CTX * _EMBED), lambda i: (0, 0)),
            pl.BlockSpec((_HID, _CTX * _EMBED), lambda i: (0, 0)),
            pl.BlockSpec((1, _HID), lambda i: (0, 0)),
            *w2_specs,
            *b2_specs,
        ],
        out_specs=pl.BlockSpec((1, _PAD_N), lambda i: (0, 0)),
        out_shape=jax.ShapeDtypeStruct((1, _VOCAB), jnp.float32),
        scratch_shapes=[
            pltpu.VMEM((1, _HID), jnp.float32),
            pltpu.VMEM((1, 128), jnp.float32),
            pltpu.VMEM((1, 128), jnp.float32),
        ],
        interpret=interpret,
    )(embeds, W1, b1_2d, *([W2] * _NSTREAM), *([b2_2d] * _NSTREAM))


def kernel(inputs, emb_table, W1, b1, W2, b2):
    idx = jnp.pad(inputs.astype(jnp.int32), (0, _CTX_PAD - _CTX))
    rows = _make_sc_gather()(idx, emb_table)
    embeds = rows[:_CTX].reshape(1, _CTX * _EMBED)
    return _mlp_call(embeds, W1, b1.reshape(1, _HID), W2,
                     b2.reshape(1, _VOCAB))


# speedup vs baseline: 1.8291x; 1.7123x over previous
import functools

import jax
import jax.numpy as jnp
from jax import lax
from jax.experimental import pallas as pl
from jax.experimental.pallas import tpu as pltpu
from jax.experimental.pallas import tpu_sc as plsc

_VOCAB = 100000
_EMBED = 64
_CTX = 50
_HID = 128
_CTX_PAD = 64

_NSTREAM = 4
_BLK = 3200
_NBLK = 8
_PAD_N = _NSTREAM * _NBLK * _BLK
_NEG = -1e30


@functools.cache
def _make_sc_gather():
    @functools.partial(
        pl.kernel,
        out_type=jax.ShapeDtypeStruct((_CTX_PAD, _EMBED), jnp.float32),
        mesh=plsc.VectorSubcoreMesh(core_axis_name="c", subcore_axis_name="s"),
        scratch_types=[
            pltpu.VMEM((_CTX_PAD,), jnp.int32),
            pltpu.VMEM((_CTX_PAD, _EMBED), jnp.float32),
            pltpu.SemaphoreType.DMA,
        ],
        compiler_params=pltpu.CompilerParams(use_tc_tiling_on_sc=False),
    )
    def _sc_gather(idx_hbm, table_hbm, out_hbm, idx_v, rows_v, sem):
        cid = lax.axis_index("c")
        sid = lax.axis_index("s")

        @pl.when((cid == 0) & (sid == 0))
        def _():
            pltpu.sync_copy(idx_hbm, idx_v)
            pltpu.async_copy(table_hbm.at[idx_v], rows_v, sem).wait()
            pltpu.sync_copy(rows_v, out_hbm)

    return _sc_gather


def _mlp_body(emb_ref, w1_ref, b1_ref, *refs):
    w2_refs = refs[:_NSTREAM]
    b2_refs = refs[_NSTREAM:2 * _NSTREAM]
    out_ref = refs[2 * _NSTREAM]
    h_ref, vm_ref, vs_ref = refs[2 * _NSTREAM + 1:]
    i = pl.program_id(0)

    @pl.when(i == 0)
    def _init():
        h = lax.dot_general(emb_ref[...], w1_ref[...],
                            (((1,), (1,)), ((), ())),
                            preferred_element_type=jnp.float32)
        h_ref[...] = jnp.maximum(h + b1_ref[...], 0.0)
        vm_ref[...] = jnp.full_like(vm_ref, _NEG)
        vs_ref[...] = jnp.zeros_like(vs_ref)

    for s in range(_NSTREAM):
        base = (s * _NBLK + i) * _BLK
        logits = lax.dot_general(h_ref[...], w2_refs[s][...],
                                 (((1,), (1,)), ((), ())),
                                 preferred_element_type=jnp.float32)
        logits = logits + b2_refs[s][...]
        out_ref[:, pl.ds(base, _BLK)] = logits

        col = base + lax.broadcasted_iota(jnp.int32, (1, _BLK), 1)
        lg = jnp.where(col < _VOCAB, logits, _NEG)
        bmax = jnp.max(lg, axis=1, keepdims=True)
        psum = jnp.sum(jnp.exp(lg - bmax), axis=1, keepdims=True)
        bm = jnp.broadcast_to(bmax, (1, 128))
        ps = jnp.broadcast_to(psum, (1, 128))
        vm_old = vm_ref[...]
        vm_new = jnp.maximum(vm_old, bm)
        vs_ref[...] = (vs_ref[...] * jnp.exp(vm_old - vm_new)
                       + ps * jnp.exp(bm - vm_new))
        vm_ref[...] = vm_new

    @pl.when(i == _NBLK - 1)
    def _fin():
        lse = (vm_ref[0:1, 0:1]
               + jnp.log(vs_ref[0:1, 0:1]))
        out_ref[...] = out_ref[...] - lse


def _mlp_call(embeds, W1, b1_2d, W2, b2_2d, interpret=False):
    w2_specs = [
        pl.BlockSpec((_BLK, _HID), lambda i, s=s: (s * _NBLK + i, 0))
        for s in range(_NSTREAM)
    ]
    b2_specs = [
        pl.BlockSpec((1, _BLK), lambda i, s=s: (0, s * _NBLK + i))
        for s in range(_NSTREAM)
    ]
    return pl.pallas_call(
        _mlp_body,
        grid=(_NBLK,),
        in_specs=[
            pl.BlockSpec((1, _CTX * _EMBED), lambda i: (0, 0)),
            pl.BlockSpec((_HID, _CTX * _EMBED), lambda i: (0, 0)),
            pl.BlockSpec((1, _HID), lambda i: (0, 0)),
            *w2_specs,
            *b2_specs,
        ],
        out_specs=pl.BlockSpec((1, _PAD_N), lambda i: (0, 0)),
        out_shape=jax.ShapeDtypeStruct((1, _VOCAB), jnp.float32),
        scratch_shapes=[
            pltpu.VMEM((1, _HID), jnp.float32),
            pltpu.VMEM((1, 128), jnp.float32),
            pltpu.VMEM((1, 128), jnp.float32),
        ],
        interpret=interpret,
    )(embeds, W1, b1_2d, *([W2] * _NSTREAM), *([b2_2d] * _NSTREAM))


def _tc_gather_body(idx_ref, table_ref, out_ref, sem):
    copies = [
        pltpu.make_async_copy(table_ref.at[idx_ref[c]], out_ref.at[c], sem)
        for c in range(_CTX)
    ]
    for cp in copies:
        cp.start()
    for cp in copies:
        cp.wait()


def _tc_gather(idx, emb_table):
    return pl.pallas_call(
        _tc_gather_body,
        in_specs=[
            pl.BlockSpec(memory_space=pltpu.SMEM),
            pl.BlockSpec(memory_space=pl.ANY),
        ],
        out_specs=pl.BlockSpec(memory_space=pltpu.VMEM),
        out_shape=jax.ShapeDtypeStruct((_CTX, _EMBED), jnp.float32),
        scratch_shapes=[pltpu.SemaphoreType.DMA],
    )(idx, emb_table)


def kernel(inputs, emb_table, W1, b1, W2, b2):
    idx = inputs.astype(jnp.int32)
    rows = _tc_gather(idx, emb_table)
    embeds = rows.reshape(1, _CTX * _EMBED)
    return _mlp_call(embeds, W1, b1.reshape(1, _HID), W2,
                     b2.reshape(1, _VOCAB))
